# Initial kernel scaffold; baseline (speedup 1.0000x reference)
import jax
import jax.numpy as jnp
from jax.experimental import pallas as pl


def kernel(idx, center_w, width_w):
    raise NotImplementedError("write your pallas kernel here")


# speedup vs baseline: 29.1033x; 29.1033x over previous
import jax
import jax.numpy as jnp
from jax import lax
from jax.experimental import pallas as pl
from jax.experimental.pallas import tpu as pltpu
from jax.experimental.pallas import tpu_sc as plsc

_NUM_WORKERS = 32
_LANES = 16
_TABLE_PAD = 1024


def _sc_lookup_body(bpw):
    nvec = bpw // _LANES

    def body(cent_hbm, wid_hbm, idx_hbm, out_hbm, cent_v, wid_v, idx_v, out_v):
        w = lax.axis_index("s") * 2 + lax.axis_index("c")
        base = w * bpw
        pltpu.sync_copy(cent_hbm, cent_v)
        pltpu.sync_copy(wid_hbm, wid_v)
        pltpu.sync_copy(idx_hbm.at[pl.ds(base, bpw)], idx_v)
        lane2 = 2 * lax.iota(jnp.int32, _LANES)

        def step(i, carry):
            ids = idx_v[pl.ds(i * _LANES, _LANES)]
            c = plsc.load_gather(cent_v, [ids])
            s = plsc.load_gather(wid_v, [ids])
            o = i * (2 * _LANES) + lane2
            plsc.store_scatter(out_v, [o], c)
            plsc.store_scatter(out_v, [o + 1], s)
            return carry

        lax.fori_loop(0, nvec, step, 0)
        pltpu.sync_copy(out_v, out_hbm.at[pl.ds(2 * base, 2 * bpw)])

    return body


def kernel(idx, center_w, width_w):
    b, l = idx.shape
    n = b * l
    bpw = n // _NUM_WORKERS
    cent = jnp.pad(center_w[:, 0], (0, _TABLE_PAD - center_w.shape[0]))
    wid = jnp.pad(width_w[:, 0], (0, _TABLE_PAD - width_w.shape[0]))
    idx_flat = idx.reshape(n)
    mesh = plsc.VectorSubcoreMesh(core_axis_name="c", subcore_axis_name="s")
    f = pl.kernel(
        _sc_lookup_body(bpw),
        out_type=jax.ShapeDtypeStruct((2 * n,), jnp.float32),
        mesh=mesh,
        compiler_params=pltpu.CompilerParams(needs_layout_passes=False),
        scratch_types=[
            pltpu.VMEM((_TABLE_PAD,), jnp.float32),
            pltpu.VMEM((_TABLE_PAD,), jnp.float32),
            pltpu.VMEM((bpw,), jnp.int32),
            pltpu.VMEM((2 * bpw,), jnp.float32),
        ],
    )
    out = f(cent, wid, idx_flat)
    return out.reshape(b, l, 2)


# speedup vs baseline: 296.5838x; 10.1907x over previous
import jax
import jax.numpy as jnp
from jax import lax
from jax.experimental import pallas as pl
from jax.experimental.pallas import tpu as pltpu
from jax.experimental.pallas import tpu_sc as plsc

_NUM_WORKERS = 32
_LANES = 16
_TABLE_PAD = 1024


def _sc_lookup_body(nl, bpw):
    tpw = bpw // nl // 128

    def body(cent_hbm, wid_hbm, idx_hbm, out_hbm, cent_v, wid_v, idx_v, out_v):
        w = lax.axis_index("s") * 2 + lax.axis_index("c")
        base = w * bpw
        pltpu.sync_copy(cent_hbm, cent_v)
        pltpu.sync_copy(wid_hbm, wid_v)
        pltpu.sync_copy(idx_hbm.at[pl.ds(base, bpw)], idx_v)
        mstride = nl * lax.iota(jnp.int32, _LANES)

        def tile_step(tp, carry):
            def l_step(l, carry2):
                for v in range(128 // _LANES):
                    p = tp * (128 * nl) + v * (_LANES * nl) + l + mstride
                    ids = plsc.load_gather(idx_v, [p])
                    cv = plsc.load_gather(cent_v, [ids])
                    wv = plsc.load_gather(wid_v, [ids])
                    out_v[l, 0, pl.ds(v * _LANES, _LANES)] = cv
                    out_v[l, 1, pl.ds(v * _LANES, _LANES)] = wv
                return carry2

            lax.fori_loop(0, nl, l_step, 0)
            pltpu.sync_copy(out_v, out_hbm.at[:, w * tpw + tp])
            return carry

        lax.fori_loop(0, tpw, tile_step, 0)

    return body


def kernel(idx, center_w, width_w):
    b, nl = idx.shape
    n = b * nl
    bpw = n // _NUM_WORKERS
    nt = b // 128
    cent = jnp.pad(center_w[:, 0], (0, _TABLE_PAD - center_w.shape[0]))
    wid = jnp.pad(width_w[:, 0], (0, _TABLE_PAD - width_w.shape[0]))
    idx_flat = idx.reshape(n)
    mesh = plsc.VectorSubcoreMesh(core_axis_name="c", subcore_axis_name="s")
    f = pl.kernel(
        _sc_lookup_body(nl, bpw),
        out_type=jax.ShapeDtypeStruct((nl, nt, 2, 128), jnp.float32),
        mesh=mesh,
        compiler_params=pltpu.CompilerParams(needs_layout_passes=False),
        scratch_types=[
            pltpu.VMEM((_TABLE_PAD,), jnp.float32),
            pltpu.VMEM((_TABLE_PAD,), jnp.float32),
            pltpu.VMEM((bpw,), jnp.int32),
            pltpu.VMEM((nl, 2, 128), jnp.float32),
        ],
    )
    out = f(cent, wid, idx_flat)
    return out.transpose(1, 3, 0, 2).reshape(b, nl, 2)


# speedup vs baseline: 382.3546x; 1.2892x over previous
import jax
import jax.numpy as jnp
from jax import lax
from jax.experimental import pallas as pl
from jax.experimental.pallas import tpu as pltpu
from jax.experimental.pallas import tpu_sc as plsc

_NUM_WORKERS = 32
_LANES = 16
_TABLE_PAD = 1024


def _sc_lookup_body(nl, bpw):
    tpw = bpw // nl // 128

    def body(cent_hbm, wid_hbm, idx_hbm, out_hbm, cent_v, wid_v, idx_v, out_v,
             sem0, sem1):
        w = lax.axis_index("s") * 2 + lax.axis_index("c")
        base = w * bpw
        pltpu.sync_copy(cent_hbm, cent_v)
        pltpu.sync_copy(wid_hbm, wid_v)
        pltpu.sync_copy(idx_hbm.at[pl.ds(base, bpw)], idx_v)
        mstride = nl * lax.iota(jnp.int32, _LANES)
        sems = (sem0, sem1)
        copies = []
        for tp in range(tpw):
            buf = tp % 2
            if tp >= 2:
                copies[tp - 2].wait()

            @plsc.parallel_loop(0, nl, unroll=2)
            def l_step(l, tp=tp, buf=buf):
                for v in range(128 // _LANES):
                    p = tp * (128 * nl) + v * (_LANES * nl) + l + mstride
                    ids = plsc.load_gather(idx_v, [p])
                    cv = plsc.load_gather(cent_v, [ids])
                    wv = plsc.load_gather(wid_v, [ids])
                    out_v[buf, l, 0, pl.ds(v * _LANES, _LANES)] = cv
                    out_v[buf, l, 1, pl.ds(v * _LANES, _LANES)] = wv

            copies.append(
                pltpu.async_copy(out_v.at[buf], out_hbm.at[:, w * tpw + tp],
                                 sems[buf]))
        copies[-2].wait()
        copies[-1].wait()

    return body


def kernel(idx, center_w, width_w):
    b, nl = idx.shape
    n = b * nl
    bpw = n // _NUM_WORKERS
    nt = b // 128
    cent = jnp.pad(center_w[:, 0], (0, _TABLE_PAD - center_w.shape[0]))
    wid = jnp.pad(width_w[:, 0], (0, _TABLE_PAD - width_w.shape[0]))
    idx_flat = idx.reshape(n)
    mesh = plsc.VectorSubcoreMesh(core_axis_name="c", subcore_axis_name="s")
    f = pl.kernel(
        _sc_lookup_body(nl, bpw),
        out_type=jax.ShapeDtypeStruct((nl, nt, 2, 128), jnp.float32),
        mesh=mesh,
        compiler_params=pltpu.CompilerParams(needs_layout_passes=False),
        scratch_types=[
            pltpu.VMEM((_TABLE_PAD,), jnp.float32),
            pltpu.VMEM((_TABLE_PAD,), jnp.float32),
            pltpu.VMEM((bpw,), jnp.int32),
            pltpu.VMEM((2, nl, 2, 128), jnp.float32),
            pltpu.SemaphoreType.DMA,
            pltpu.SemaphoreType.DMA,
        ],
    )
    out = f(cent, wid, idx_flat)
    return out.transpose(1, 3, 0, 2).reshape(b, nl, 2)


# speedup vs baseline: 526.6449x; 1.3774x over previous
import jax
import jax.numpy as jnp
from jax import lax
from jax.experimental import pallas as pl
from jax.experimental.pallas import tpu as pltpu
from jax.experimental.pallas import tpu_sc as plsc

_NUM_WORKERS = 32
_LANES = 16
_TABLE_PAD = 1024


def _sc_lookup_body(nl, tpw):
    nlt = (nl + 7) // 8

    def body(cent_hbm, wid_hbm, idx_hbm, out_hbm, cent_v, wid_v, idx_s0,
             idx_s1, out_v, isem0, isem1, osem0, osem1):
        w = lax.axis_index("s") * 2 + lax.axis_index("c")
        t0 = w * tpw
        pltpu.sync_copy(cent_hbm, cent_v)
        pltpu.sync_copy(wid_hbm, wid_v)
        iota = lax.iota(jnp.int32, _LANES)
        idx_bufs = (idx_s0, idx_s1)
        isems = (isem0, isem1)
        osems = (osem0, osem1)
        in_copies = [
            pltpu.async_copy(idx_hbm.at[:, t0], idx_bufs[0], isems[0])
        ]
        out_copies = []
        for tp in range(tpw):
            ibuf = tp % 2
            if tp + 1 < tpw:
                in_copies.append(
                    pltpu.async_copy(idx_hbm.at[:, t0 + tp + 1],
                                     idx_bufs[1 - ibuf], isems[1 - ibuf]))
            in_copies[tp].wait()
            if tp >= 2:
                out_copies[tp - 2].wait()

            @plsc.parallel_loop(0, nl, unroll=2)
            def l_step(l, ibuf=ibuf, obuf=ibuf):
                ltv = jnp.full((_LANES,), 0, jnp.int32) + (l >> 3)
                lsv = jnp.full((_LANES,), 0, jnp.int32) + (l & 7)
                for v in range(128 // _LANES):
                    mv = v * _LANES + iota
                    ids = plsc.load_gather(idx_bufs[ibuf], [ltv, lsv, mv])
                    cv = plsc.load_gather(cent_v, [ids])
                    wv = plsc.load_gather(wid_v, [ids])
                    out_v[obuf, l, 0, pl.ds(v * _LANES, _LANES)] = cv
                    out_v[obuf, l, 1, pl.ds(v * _LANES, _LANES)] = wv

            out_copies.append(
                pltpu.async_copy(out_v.at[ibuf], out_hbm.at[:, t0 + tp],
                                 osems[ibuf]))
        out_copies[-2].wait()
        out_copies[-1].wait()

    return body


def kernel(idx, center_w, width_w):
    b, nl = idx.shape
    nt = b // 128
    tpw = nt // _NUM_WORKERS
    nlt = (nl + 7) // 8
    cent = jnp.pad(center_w[:, 0], (0, _TABLE_PAD - center_w.shape[0]))
    wid = jnp.pad(width_w[:, 0], (0, _TABLE_PAD - width_w.shape[0]))
    idx_p = jnp.pad(idx, ((0, 0), (0, nlt * 8 - nl)))
    idx4 = idx_p.T.reshape(nlt, 8, nt, 128).transpose(0, 2, 1, 3)
    mesh = plsc.VectorSubcoreMesh(core_axis_name="c", subcore_axis_name="s")
    f = pl.kernel(
        _sc_lookup_body(nl, tpw),
        out_type=jax.ShapeDtypeStruct((nl, nt, 2, 128), jnp.float32),
        mesh=mesh,
        compiler_params=pltpu.CompilerParams(needs_layout_passes=False),
        scratch_types=[
            pltpu.VMEM((_TABLE_PAD,), jnp.float32),
            pltpu.VMEM((_TABLE_PAD,), jnp.float32),
            pltpu.VMEM((nlt, 8, 128), jnp.int32),
            pltpu.VMEM((nlt, 8, 128), jnp.int32),
            pltpu.VMEM((2, nl, 2, 128), jnp.float32),
            pltpu.SemaphoreType.DMA,
            pltpu.SemaphoreType.DMA,
            pltpu.SemaphoreType.DMA,
            pltpu.SemaphoreType.DMA,
        ],
    )
    out = f(cent, wid, idx4)
    return out.transpose(1, 3, 0, 2).reshape(b, nl, 2)


# speedup vs baseline: 559.4090x; 1.0622x over previous
import jax
import jax.numpy as jnp
from jax import lax
from jax.experimental import pallas as pl
from jax.experimental.pallas import tpu as pltpu
from jax.experimental.pallas import tpu_sc as plsc

_NUM_WORKERS = 32
_LANES = 16
_TABLE_PAD = 1024


def _sc_lookup_body(nl, tpw):
    nlt = (nl + 7) // 8

    def body(cent_hbm, wid_hbm, idx_hbm, out_hbm, cent_v, wid_v, idx_s0,
             idx_s1, out_v, isem0, isem1, osem0, osem1):
        w = lax.axis_index("s") * 2 + lax.axis_index("c")
        t0 = w * tpw
        pltpu.sync_copy(cent_hbm, cent_v)
        pltpu.sync_copy(wid_hbm, wid_v)
        idx_bufs = (idx_s0, idx_s1)
        isems = (isem0, isem1)
        osems = (osem0, osem1)
        in_copies = [
            pltpu.async_copy(idx_hbm.at[:, t0], idx_bufs[0], isems[0])
        ]
        out_copies = []
        for tp in range(tpw):
            ibuf = tp % 2
            if tp + 1 < tpw:
                in_copies.append(
                    pltpu.async_copy(idx_hbm.at[:, t0 + tp + 1],
                                     idx_bufs[1 - ibuf], isems[1 - ibuf]))
            in_copies[tp].wait()
            if tp >= 2:
                out_copies[tp - 2].wait()

            @plsc.parallel_loop(0, nl, unroll=2)
            def l_step(l, ibuf=ibuf, obuf=ibuf):
                lt = l >> 3
                ls = l & 7
                for v in range(128 // _LANES):
                    ids = idx_bufs[ibuf][lt, ls, pl.ds(v * _LANES, _LANES)]
                    cv = plsc.load_gather(cent_v, [ids])
                    wv = plsc.load_gather(wid_v, [ids])
                    out_v[obuf, l, 0, pl.ds(v * _LANES, _LANES)] = cv
                    out_v[obuf, l, 1, pl.ds(v * _LANES, _LANES)] = wv

            out_copies.append(
                pltpu.async_copy(out_v.at[ibuf], out_hbm.at[:, t0 + tp],
                                 osems[ibuf]))
        out_copies[-2].wait()
        out_copies[-1].wait()

    return body


def kernel(idx, center_w, width_w):
    b, nl = idx.shape
    nt = b // 128
    tpw = nt // _NUM_WORKERS
    nlt = (nl + 7) // 8
    nv = center_w.shape[0]
    cent = center_w[:, 0]
    wid = width_w[:, 0]
    idx_p = jnp.pad(idx, ((0, 0), (0, nlt * 8 - nl)))
    idx4 = idx_p.T.reshape(nlt, 8, nt, 128).transpose(0, 2, 1, 3)
    mesh = plsc.VectorSubcoreMesh(core_axis_name="c", subcore_axis_name="s")
    f = pl.kernel(
        _sc_lookup_body(nl, tpw),
        out_type=jax.ShapeDtypeStruct((nl, nt, 2, 128), jnp.float32),
        mesh=mesh,
        compiler_params=pltpu.CompilerParams(needs_layout_passes=False),
        scratch_types=[
            pltpu.VMEM((nv,), jnp.float32),
            pltpu.VMEM((nv,), jnp.float32),
            pltpu.VMEM((nlt, 8, 128), jnp.int32),
            pltpu.VMEM((nlt, 8, 128), jnp.int32),
            pltpu.VMEM((2, nl, 2, 128), jnp.float32),
            pltpu.SemaphoreType.DMA,
            pltpu.SemaphoreType.DMA,
            pltpu.SemaphoreType.DMA,
            pltpu.SemaphoreType.DMA,
        ],
    )
    out = f(cent, wid, idx4)
    return out.transpose(1, 3, 0, 2).reshape(b, nl, 2)
